# baseline (speedup 1.0000x reference)
import functools

import jax
import jax.numpy as jnp
from jax.experimental import pallas as pl
from jax.experimental.pallas import tpu as pltpu

_K = 1024
_D = 64
_TB = 512
_COMMIT = 0.25


def _rowsum_sq(xb):
    sq = xb * xb
    acc = sq[:, 0:8]
    for j in range(1, 8):
        acc = acc + sq[:, 8 * j:8 * j + 8]
    a4 = acc[:, 0:4] + acc[:, 4:8]
    a2 = a4[:, 0:2] + a4[:, 2:4]
    return a2[:, 0:1] + a2[:, 1:2]


def _colsum_sq_t(wt):
    sq = wt * wt
    acc = sq[0:8, :]
    for j in range(1, 8):
        acc = acc + sq[8 * j:8 * j + 8, :]
    a4 = acc[0:4, :] + acc[4:8, :]
    a2 = a4[0:2, :] + a4[2:4, :]
    return a2[0:1, :] + a2[1:2, :]


def _vq_body(x_ref, w_ref, wt_ref, qst_ref, idx_ref, loss_ref, perp_ref,
             counts_ref, ssum_ref, *, grid_n, total_tokens):
    i = pl.program_id(0)

    @pl.when(i == 0)
    def _init():
        ssum_ref[0] = 0.0
        counts_ref[...] = jnp.zeros_like(counts_ref)

    xb = x_ref[...]
    w = w_ref[...]
    rowsum = _rowsum_sq(xb)
    colsum = _colsum_sq_t(wt_ref[...])
    mm = jax.lax.dot_general(xb, w, (((1,), (1,)), ((), ())),
                             preferred_element_type=jnp.float32)
    d = (rowsum + colsum) - 2.0 * mm
    d = jnp.maximum(d, 0.0)
    dmin = jnp.min(d, axis=1, keepdims=True)
    iota = jax.lax.broadcasted_iota(jnp.int32, (_TB, _K), 1)
    idx = jnp.min(jnp.where(d == dmin, iota, _K), axis=1).astype(jnp.int32)

    onehot = (iota == idx[:, None]).astype(jnp.float32)
    q = jax.lax.dot_general(onehot, w, (((1,), (0,)), ((), ())),
                            preferred_element_type=jnp.float32,
                            precision=jax.lax.Precision.HIGHEST)
    qst_ref[...] = xb + (q - xb)
    idx_ref[...] = idx.reshape(1, 1, _TB)

    ssum_ref[0] += jnp.sum(dmin[:, 0])
    counts_ref[...] += jnp.sum(onehot, axis=0)[None, :]

    @pl.when(i == grid_n - 1)
    def _fini():
        loss = ssum_ref[0] / (total_tokens * _D)
        loss_ref[...] = jnp.full((1, 1), loss + _COMMIT * loss, jnp.float32)
        p = counts_ref[...] * (1.0 / total_tokens)
        ent = jnp.sum(p * jnp.log(p + 1e-10))
        perp_ref[...] = jnp.full((1, 1), jnp.exp(-ent), jnp.float32)


@jax.jit
def kernel(x, W):
    orig_shape = x.shape
    x_flat = x.reshape(-1, _D)
    total = x_flat.shape[0]
    grid_n = total // _TB

    body = functools.partial(_vq_body, grid_n=grid_n, total_tokens=total)
    qst, idx3, loss, perp = pl.pallas_call(
        body,
        grid=(grid_n,),
        in_specs=[
            pl.BlockSpec((_TB, _D), lambda i: (i, 0)),
            pl.BlockSpec((_K, _D), lambda i: (0, 0)),
            pl.BlockSpec((_D, _K), lambda i: (0, 0)),
        ],
        out_specs=[
            pl.BlockSpec((_TB, _D), lambda i: (i, 0)),
            pl.BlockSpec((1, 1, _TB), lambda i: (i, 0, 0)),
            pl.BlockSpec((1, 1), lambda i: (0, 0)),
            pl.BlockSpec((1, 1), lambda i: (0, 0)),
        ],
        out_shape=[
            jax.ShapeDtypeStruct((total, _D), jnp.float32),
            jax.ShapeDtypeStruct((grid_n, 1, _TB), jnp.int32),
            jax.ShapeDtypeStruct((1, 1), jnp.float32),
            jax.ShapeDtypeStruct((1, 1), jnp.float32),
        ],
        scratch_shapes=[
            pltpu.VMEM((1, _K), jnp.float32),
            pltpu.SMEM((1,), jnp.float32),
        ],
    )(x_flat, W, W.T)

    quantized_st = qst.reshape(orig_shape)
    encoding_indices = idx3.reshape(total)
    return (quantized_st, loss[0, 0], perp[0, 0], encoding_indices)


# speedup vs baseline: 1.1669x; 1.1669x over previous
import functools

import jax
import jax.numpy as jnp
from jax.experimental import pallas as pl
from jax.experimental.pallas import tpu as pltpu

_K = 1024
_D = 64
_TB = 1024
_COMMIT = 0.25


def _rowsum_sq(xb):
    sq = xb * xb
    acc = sq[:, 0:8]
    for j in range(1, 8):
        acc = acc + sq[:, 8 * j:8 * j + 8]
    a4 = acc[:, 0:4] + acc[:, 4:8]
    a2 = a4[:, 0:2] + a4[:, 2:4]
    return a2[:, 0:1] + a2[:, 1:2]


def _colsum_sq_t(wt):
    sq = wt * wt
    acc = sq[0:8, :]
    for j in range(1, 8):
        acc = acc + sq[8 * j:8 * j + 8, :]
    a4 = acc[0:4, :] + acc[4:8, :]
    a2 = a4[0:2, :] + a4[2:4, :]
    return a2[0:1, :] + a2[1:2, :]


def _vq_body(x_ref, w_ref, wt_ref, qst_ref, idx_ref, loss_ref, perp_ref,
             counts_ref, ssum_ref, *, grid_n, total_tokens):
    i = pl.program_id(0)

    @pl.when(i == 0)
    def _init():
        ssum_ref[0] = 0.0
        counts_ref[...] = jnp.zeros_like(counts_ref)

    xb = x_ref[...]
    w = w_ref[...]
    rowsum = _rowsum_sq(xb)
    colsum = _colsum_sq_t(wt_ref[...])
    mm = jax.lax.dot_general(xb, w, (((1,), (1,)), ((), ())),
                             preferred_element_type=jnp.float32)
    d = (rowsum + colsum) - 2.0 * mm
    d = jnp.maximum(d, 0.0)
    dmin = jnp.min(d, axis=1, keepdims=True)
    iota = jax.lax.broadcasted_iota(jnp.int32, (_TB, _K), 1)
    idx = jnp.min(jnp.where(d == dmin, iota, _K), axis=1).astype(jnp.int32)

    onehot = (iota == idx[:, None]).astype(jnp.float32)
    q = jax.lax.dot_general(onehot, w, (((1,), (0,)), ((), ())),
                            preferred_element_type=jnp.float32)
    qst_ref[...] = xb + (q - xb)
    idx_ref[...] = idx.reshape(1, 1, _TB)

    ssum_ref[0] += jnp.sum(dmin[:, 0])
    counts_ref[...] += jnp.sum(onehot, axis=0)[None, :]

    @pl.when(i == grid_n - 1)
    def _fini():
        loss = ssum_ref[0] / (total_tokens * _D)
        loss_ref[...] = jnp.full((1, 1), loss + _COMMIT * loss, jnp.float32)
        p = counts_ref[...] * (1.0 / total_tokens)
        ent = jnp.sum(p * jnp.log(p + 1e-10))
        perp_ref[...] = jnp.full((1, 1), jnp.exp(-ent), jnp.float32)


@jax.jit
def kernel(x, W):
    orig_shape = x.shape
    x_flat = x.reshape(-1, _D)
    total = x_flat.shape[0]
    grid_n = total // _TB

    body = functools.partial(_vq_body, grid_n=grid_n, total_tokens=total)
    qst, idx3, loss, perp = pl.pallas_call(
        body,
        grid=(grid_n,),
        in_specs=[
            pl.BlockSpec((_TB, _D), lambda i: (i, 0)),
            pl.BlockSpec((_K, _D), lambda i: (0, 0)),
            pl.BlockSpec((_D, _K), lambda i: (0, 0)),
        ],
        out_specs=[
            pl.BlockSpec((_TB, _D), lambda i: (i, 0)),
            pl.BlockSpec((1, 1, _TB), lambda i: (i, 0, 0)),
            pl.BlockSpec((1, 1), lambda i: (0, 0)),
            pl.BlockSpec((1, 1), lambda i: (0, 0)),
        ],
        out_shape=[
            jax.ShapeDtypeStruct((total, _D), jnp.float32),
            jax.ShapeDtypeStruct((grid_n, 1, _TB), jnp.int32),
            jax.ShapeDtypeStruct((1, 1), jnp.float32),
            jax.ShapeDtypeStruct((1, 1), jnp.float32),
        ],
        scratch_shapes=[
            pltpu.VMEM((1, _K), jnp.float32),
            pltpu.SMEM((1,), jnp.float32),
        ],
    )(x_flat, W, W.T)

    quantized_st = qst.reshape(orig_shape)
    encoding_indices = idx3.reshape(total)
    return (quantized_st, loss[0, 0], perp[0, 0], encoding_indices)


# speedup vs baseline: 1.2184x; 1.0441x over previous
import functools

import jax
import jax.numpy as jnp
from jax.experimental import pallas as pl
from jax.experimental.pallas import tpu as pltpu

_K = 1024
_D = 64
_TB = 1024
_COMMIT = 0.25


def _rowsum_sq(xb):
    sq = xb * xb
    acc = sq[:, 0:8]
    for j in range(1, 8):
        acc = acc + sq[:, 8 * j:8 * j + 8]
    a4 = acc[:, 0:4] + acc[:, 4:8]
    a2 = a4[:, 0:2] + a4[:, 2:4]
    return a2[:, 0:1] + a2[:, 1:2]


def _colsum_sq_t(wt):
    sq = wt * wt
    acc = sq[0:8, :]
    for j in range(1, 8):
        acc = acc + sq[8 * j:8 * j + 8, :]
    a4 = acc[0:4, :] + acc[4:8, :]
    a2 = a4[0:2, :] + a4[2:4, :]
    return a2[0:1, :] + a2[1:2, :]


def _vq_body(x_ref, w_ref, wt_ref, qst_ref, idx_ref, loss_ref, perp_ref,
             counts_ref, ssum_ref, *, grid_n, total_tokens):
    i = pl.program_id(0)

    @pl.when(i == 0)
    def _init():
        ssum_ref[0] = 0.0
        counts_ref[...] = jnp.zeros_like(counts_ref)

    xb = x_ref[...]
    w = w_ref[...]
    rowsum = _rowsum_sq(xb)
    colsum = _colsum_sq_t(wt_ref[...])
    mm = jax.lax.dot_general(xb, w, (((1,), (1,)), ((), ())),
                             preferred_element_type=jnp.float32)
    d = (rowsum + colsum) - 2.0 * mm
    d = jnp.maximum(d, 0.0)
    dmin = jnp.min(d, axis=1, keepdims=True)
    iota = jax.lax.broadcasted_iota(jnp.int32, (_TB, _K), 1)
    idx = jnp.min(jnp.where(d == dmin, iota, _K), axis=1).astype(jnp.int32)

    onehot = (iota == idx[:, None]).astype(jnp.float32).astype(jnp.bfloat16)
    q = jax.lax.dot_general(onehot, w.astype(jnp.bfloat16),
                            (((1,), (0,)), ((), ())),
                            preferred_element_type=jnp.float32)
    qst_ref[...] = xb + (q - xb)
    idx_ref[...] = idx[:, None]

    ssum_ref[0] += jnp.sum(dmin[:, 0])
    ones_row = jnp.ones((1, _TB), jnp.bfloat16)
    counts_ref[...] += jax.lax.dot_general(
        ones_row, onehot, (((1,), (0,)), ((), ())),
        preferred_element_type=jnp.float32)

    @pl.when(i == grid_n - 1)
    def _fini():
        loss = ssum_ref[0] / (total_tokens * _D)
        loss_ref[...] = jnp.full((1, 1), loss + _COMMIT * loss, jnp.float32)
        p = counts_ref[...] * (1.0 / total_tokens)
        ent = jnp.sum(p * jnp.log(p + 1e-10))
        perp_ref[...] = jnp.full((1, 1), jnp.exp(-ent), jnp.float32)


@jax.jit
def kernel(x, W):
    orig_shape = x.shape
    x_flat = x.reshape(-1, _D)
    total = x_flat.shape[0]
    grid_n = total // _TB

    body = functools.partial(_vq_body, grid_n=grid_n, total_tokens=total)
    qst, idx2, loss, perp = pl.pallas_call(
        body,
        grid=(grid_n,),
        in_specs=[
            pl.BlockSpec((_TB, _D), lambda i: (i, 0)),
            pl.BlockSpec((_K, _D), lambda i: (0, 0)),
            pl.BlockSpec((_D, _K), lambda i: (0, 0)),
        ],
        out_specs=[
            pl.BlockSpec((_TB, _D), lambda i: (i, 0)),
            pl.BlockSpec((_TB, 1), lambda i: (i, 0)),
            pl.BlockSpec((1, 1), lambda i: (0, 0)),
            pl.BlockSpec((1, 1), lambda i: (0, 0)),
        ],
        out_shape=[
            jax.ShapeDtypeStruct((total, _D), jnp.float32),
            jax.ShapeDtypeStruct((total, 1), jnp.int32),
            jax.ShapeDtypeStruct((1, 1), jnp.float32),
            jax.ShapeDtypeStruct((1, 1), jnp.float32),
        ],
        scratch_shapes=[
            pltpu.VMEM((1, _K), jnp.float32),
            pltpu.SMEM((1,), jnp.float32),
        ],
    )(x_flat, W, W.T)

    quantized_st = qst.reshape(orig_shape)
    encoding_indices = idx2.reshape(total)
    return (quantized_st, loss[0, 0], perp[0, 0], encoding_indices)
